# Initial kernel scaffold; baseline (speedup 1.0000x reference)
import jax
import jax.numpy as jnp
from jax.experimental import pallas as pl


def kernel(input, table):
    raise NotImplementedError("write your pallas kernel here")


# speedup vs baseline: 4.8730x; 4.8730x over previous
import functools

import jax
import jax.numpy as jnp
from jax import lax
from jax.experimental import pallas as pl
from jax.experimental.pallas import tpu as pltpu
from jax.experimental.pallas import tpu_sc as plsc

EMBEDDING_DIM = 32


@functools.partial(jax.jit, static_argnums=(0, 1, 2))
def _gather_call(B, D, C, table, idx):
    info = plsc.get_sparse_core_info()
    NC, NS = info.num_cores, info.num_subcores
    NW = NC * NS
    b_per_w = B // NW
    n_chunks = b_per_w // C
    assert n_chunks % 2 == 0
    mesh = plsc.VectorSubcoreMesh(core_axis_name="c", subcore_axis_name="s")

    @functools.partial(
        pl.kernel,
        mesh=mesh,
        out_type=jax.ShapeDtypeStruct((B, D), jnp.float32),
        scratch_types=[
            pltpu.VMEM((2, C), jnp.int32),
            pltpu.VMEM((2, C, D), jnp.float32),
            pltpu.SemaphoreType.DMA,
            pltpu.SemaphoreType.DMA,
        ],
        compiler_params=pltpu.CompilerParams(use_tc_tiling_on_sc=False),
    )
    def k(table_hbm, idx_hbm, out_hbm, idx_v, rows_v, gsem0, gsem1):
        gsems = (gsem0, gsem1)
        wid = lax.axis_index("s") * NC + lax.axis_index("c")
        base = wid * b_per_w

        def start(g, b):
            off = base + g * C
            pltpu.sync_copy(idx_hbm.at[pl.ds(off, C)], idx_v.at[b])
            pltpu.async_copy(table_hbm.at[idx_v.at[b]], rows_v.at[b], gsems[b])

        def wait(b):
            pltpu.make_async_copy(
                table_hbm.at[idx_v.at[b]], rows_v.at[b], gsems[b]
            ).wait()

        start(0, 0)
        start(1, 1)

        def body(i, carry):
            g0 = i * 2
            for b in range(2):
                g = g0 + b
                wait(b)
                pltpu.sync_copy(rows_v.at[b], out_hbm.at[pl.ds(base + g * C, C)])

                @pl.when(g + 2 < n_chunks)
                def _():
                    start(g + 2, b)

            return carry

        lax.fori_loop(0, n_chunks // 2, body, 0)

    return k(table, idx)


def kernel(input, table):
    B = input.shape[0] * input.shape[1]
    idx = input.reshape(B).astype(jnp.int32)
    out = _gather_call(B, EMBEDDING_DIM, 512, table, idx)
    return out.reshape(input.shape + (EMBEDDING_DIM,))
